# baseline (speedup 1.0000x reference)
import jax
import jax.numpy as jnp
from jax import lax
from jax.experimental import pallas as pl
from jax.experimental.pallas import tpu as pltpu, tpu_sc as plsc

_B = 1024
_D = 32
_L = 16
_NC = 2
_NS = 16
_NW = _NC * _NS
_RPW = _B // _NW
_NCHUNK = _B // _L


def _newton_sqrt(x):
    i = lax.bitcast_convert_type(x, jnp.int32)
    i = jnp.int32(0x5F3759DF) - lax.shift_right_arithmetic(i, 1)
    y = lax.bitcast_convert_type(i, jnp.float32)
    for _ in range(3):
        y = y * (1.5 - 0.5 * x * y * y)
    return x * y


def _sc_body(et_hbm, erep_hbm, lab_hbm, labrep_hbm, out_hbm,
             et_v, erep_v, lab_v, labrep_v, out_v):
    wid = lax.axis_index("s") * _NC + lax.axis_index("c")
    base = wid * _RPW
    pltpu.sync_copy(et_hbm, et_v)
    pltpu.sync_copy(erep_hbm.at[pl.ds(base * _D * _L, _RPW * _D * _L)],
                    erep_v)
    pltpu.sync_copy(lab_hbm, lab_v)
    pltpu.sync_copy(labrep_hbm.at[pl.ds(base * _L, _RPW * _L)], labrep_v)

    def row_body(r, _):
        splats = [erep_v[pl.ds((r * _D + d) * _L, _L)] for d in range(_D)]
        lab_r = labrep_v[pl.ds(r * _L, _L)]

        def chunk_body(cc, _):
            c0 = cc * _L
            acc = jnp.zeros((_L,), jnp.float32)
            for d in range(_D):
                v = et_v[d, pl.ds(c0, _L)] - splats[d]
                acc = acc + v * v
            dist = _newton_sqrt(acc + 1e-12)
            eq = jnp.where(lab_v[pl.ds(c0, _L)] == lab_r, 1.0, 0.0)
            diff = eq - dist
            out_v[r, pl.ds(c0, _L)] = diff * diff
            return 0

        lax.fori_loop(0, _NCHUNK, chunk_body, 0)
        return 0

    lax.fori_loop(0, _RPW, row_body, 0)
    pltpu.sync_copy(out_v, out_hbm.at[pl.ds(base, _RPW)])


def kernel(embeddings, labels):
    labels = labels.astype(jnp.int32)
    et = embeddings.T.copy()
    erep = jnp.broadcast_to(embeddings[:, :, None], (_B, _D, _L)).reshape(-1)
    labrep = jnp.broadcast_to(labels[:, None], (_B, _L)).reshape(-1)
    mesh = plsc.VectorSubcoreMesh(
        core_axis_name="c", subcore_axis_name="s",
        num_cores=_NC, num_subcores=_NS)
    out = pl.kernel(
        _sc_body,
        out_type=jax.ShapeDtypeStruct((_B, _B), jnp.float32),
        mesh=mesh,
        scratch_types=[
            pltpu.VMEM((_D, _B), jnp.float32),
            pltpu.VMEM((_RPW * _D * _L,), jnp.float32),
            pltpu.VMEM((_B,), jnp.int32),
            pltpu.VMEM((_RPW * _L,), jnp.int32),
            pltpu.VMEM((_RPW, _B), jnp.float32),
        ],
    )(et, erep, labels, labrep)
    return out.reshape(-1)


# speedup vs baseline: 1.5129x; 1.5129x over previous
import jax
import jax.numpy as jnp
from jax import lax
from jax.experimental import pallas as pl
from jax.experimental.pallas import tpu as pltpu, tpu_sc as plsc

_B = 1024
_D = 32
_L = 16
_NC = 2
_NS = 16
_NW = _NC * _NS
_RPW = _B // _NW
_NCHUNK = _B // _L


def _newton_sqrt(x):
    i = lax.bitcast_convert_type(x, jnp.int32)
    i = jnp.int32(0x5F3759DF) - lax.shift_right_arithmetic(i, 1)
    y = lax.bitcast_convert_type(i, jnp.float32)
    for _ in range(2):
        y = y * (1.5 - 0.5 * x * y * y)
    return x * y


def _sc_body(et_hbm, erep_hbm, lab_hbm, labrep_hbm, out_hbm,
             et_v, erep_v, lab_v, labrep_v, out_v):
    wid = lax.axis_index("s") * _NC + lax.axis_index("c")
    base = wid * _RPW
    pltpu.sync_copy(et_hbm, et_v)
    pltpu.sync_copy(erep_hbm.at[pl.ds(base * _D * _L, _RPW * _D * _L)],
                    erep_v)
    pltpu.sync_copy(lab_hbm, lab_v)
    pltpu.sync_copy(labrep_hbm.at[pl.ds(base * _L, _RPW * _L)], labrep_v)

    @plsc.parallel_loop(0, _RPW)
    def row_body(r):
        splats = [erep_v[pl.ds((r * _D + d) * _L, _L)] for d in range(_D)]
        lab_r = labrep_v[pl.ds(r * _L, _L)]

        @plsc.parallel_loop(0, _NCHUNK, unroll=2)
        def chunk_body(cc):
            c0 = cc * _L
            accs = [jnp.zeros((_L,), jnp.float32) for _ in range(4)]
            for d in range(_D):
                v = et_v[d, pl.ds(c0, _L)] - splats[d]
                accs[d % 4] = accs[d % 4] + v * v
            acc = (accs[0] + accs[1]) + (accs[2] + accs[3])
            dist = _newton_sqrt(acc + 1e-12)
            eq = jnp.where(lab_v[pl.ds(c0, _L)] == lab_r, 1.0, 0.0)
            diff = eq - dist
            out_v[r, pl.ds(c0, _L)] = diff * diff
    pltpu.sync_copy(out_v, out_hbm.at[pl.ds(base, _RPW)])


def kernel(embeddings, labels):
    labels = labels.astype(jnp.int32)
    et = embeddings.T.copy()
    erep = jnp.broadcast_to(embeddings[:, :, None], (_B, _D, _L)).reshape(-1)
    labrep = jnp.broadcast_to(labels[:, None], (_B, _L)).reshape(-1)
    mesh = plsc.VectorSubcoreMesh(
        core_axis_name="c", subcore_axis_name="s",
        num_cores=_NC, num_subcores=_NS)
    out = pl.kernel(
        _sc_body,
        out_type=jax.ShapeDtypeStruct((_B, _B), jnp.float32),
        mesh=mesh,
        scratch_types=[
            pltpu.VMEM((_D, _B), jnp.float32),
            pltpu.VMEM((_RPW * _D * _L,), jnp.float32),
            pltpu.VMEM((_B,), jnp.int32),
            pltpu.VMEM((_RPW * _L,), jnp.int32),
            pltpu.VMEM((_RPW, _B), jnp.float32),
        ],
    )(et, erep, labels, labrep)
    return out.reshape(-1)


# speedup vs baseline: 1.8650x; 1.2327x over previous
import jax
import jax.numpy as jnp
from jax import lax
from jax.experimental import pallas as pl
from jax.experimental.pallas import tpu as pltpu, tpu_sc as plsc

_B = 1024
_D = 32
_L = 16
_NC = 2
_NS = 16
_NW = _NC * _NS
_RPW = _B // _NW
_NCHUNK = _B // _L


def _newton_sqrt(x):
    i = lax.bitcast_convert_type(x, jnp.int32)
    i = jnp.int32(0x5F3759DF) - lax.shift_right_arithmetic(i, 1)
    y = lax.bitcast_convert_type(i, jnp.float32)
    for _ in range(2):
        y = y * (1.5 - 0.5 * x * y * y)
    return x * y


def _splat(ref, idx):
    return plsc.load_gather(ref, [jnp.full((_L,), idx, jnp.int32)])


def _sc_body(et_hbm, e_hbm, lab_hbm, out_hbm, et_v, e_v, lab_v, out_v):
    wid = lax.axis_index("s") * _NC + lax.axis_index("c")
    base = wid * _RPW
    pltpu.sync_copy(et_hbm, et_v)
    pltpu.sync_copy(e_hbm.at[pl.ds(base * _D, _RPW * _D)], e_v)
    pltpu.sync_copy(lab_hbm, lab_v)

    @plsc.parallel_loop(0, _RPW)
    def row_body(r):
        splats = [_splat(e_v, r * _D + d) for d in range(_D)]
        lab_r = _splat(lab_v, base + r)

        @plsc.parallel_loop(0, _NCHUNK, unroll=2)
        def chunk_body(cc):
            c0 = cc * _L
            accs = [jnp.zeros((_L,), jnp.float32) for _ in range(4)]
            for d in range(_D):
                v = et_v[d, pl.ds(c0, _L)] - splats[d]
                accs[d % 4] = accs[d % 4] + v * v
            acc = (accs[0] + accs[1]) + (accs[2] + accs[3])
            dist = _newton_sqrt(acc + 1e-12)
            eq = jnp.where(lab_v[pl.ds(c0, _L)] == lab_r, 1.0, 0.0)
            diff = eq - dist
            out_v[r, pl.ds(c0, _L)] = diff * diff

    pltpu.sync_copy(out_v, out_hbm.at[pl.ds(base, _RPW)])


def kernel(embeddings, labels):
    labels = labels.astype(jnp.int32)
    et = embeddings.T.copy()
    e_flat = embeddings.reshape(-1)
    mesh = plsc.VectorSubcoreMesh(
        core_axis_name="c", subcore_axis_name="s",
        num_cores=_NC, num_subcores=_NS)
    out = pl.kernel(
        _sc_body,
        out_type=jax.ShapeDtypeStruct((_B, _B), jnp.float32),
        mesh=mesh,
        compiler_params=pltpu.CompilerParams(needs_layout_passes=False),
        scratch_types=[
            pltpu.VMEM((_D, _B), jnp.float32),
            pltpu.VMEM((_RPW * _D,), jnp.float32),
            pltpu.VMEM((_B,), jnp.int32),
            pltpu.VMEM((_RPW, _B), jnp.float32),
        ],
    )(et, e_flat, labels)
    return out.reshape(-1)


# speedup vs baseline: 1.9959x; 1.0702x over previous
import jax
import jax.numpy as jnp
from jax import lax
from jax.experimental import pallas as pl
from jax.experimental.pallas import tpu as pltpu, tpu_sc as plsc

_B = 1024
_D = 32
_L = 16
_NC = 2
_NS = 16
_NW = _NC * _NS
_RPW = _B // _NW
_NCHUNK = _B // _L


def _newton_sqrt(x):
    i = lax.bitcast_convert_type(x, jnp.int32)
    i = jnp.int32(0x5F3759DF) - lax.shift_right_arithmetic(i, 1)
    y = lax.bitcast_convert_type(i, jnp.float32)
    for _ in range(2):
        y = y * (1.5 - 0.5 * x * y * y)
    return x * y


def _splat(ref, idx):
    return plsc.load_gather(ref, [jnp.full((_L,), idx, jnp.int32)])


def _sc_body(et_hbm, e_hbm, lab_hbm, out_hbm, et_v, e_v, lab_v, out_v):
    wid = lax.axis_index("s") * _NC + lax.axis_index("c")
    base = wid * _RPW
    pltpu.sync_copy(et_hbm, et_v)
    pltpu.sync_copy(e_hbm.at[pl.ds(base * _D, _RPW * _D)], e_v)
    pltpu.sync_copy(lab_hbm, lab_v)

    @plsc.parallel_loop(0, _RPW)
    def row_body(r):
        splats = [_splat(e_v, r * _D + d) for d in range(_D)]
        lab_r = _splat(lab_v, base + r)

        @plsc.parallel_loop(0, _NCHUNK, unroll=4)
        def chunk_body(cc):
            c0 = cc * _L
            accs = [jnp.zeros((_L,), jnp.float32) for _ in range(4)]
            for d in range(_D):
                v = et_v[d, pl.ds(c0, _L)] - splats[d]
                accs[d % 4] = accs[d % 4] + v * v
            acc = (accs[0] + accs[1]) + (accs[2] + accs[3])
            dist = _newton_sqrt(acc + 1e-12)
            eq = jnp.where(lab_v[pl.ds(c0, _L)] == lab_r, 1.0, 0.0)
            diff = eq - dist
            out_v[r, pl.ds(c0, _L)] = diff * diff

    pltpu.sync_copy(out_v, out_hbm.at[pl.ds(base, _RPW)])


def kernel(embeddings, labels):
    labels = labels.astype(jnp.int32)
    et = embeddings.T.copy()
    e_flat = embeddings.reshape(-1)
    mesh = plsc.VectorSubcoreMesh(
        core_axis_name="c", subcore_axis_name="s",
        num_cores=_NC, num_subcores=_NS)
    out = pl.kernel(
        _sc_body,
        out_type=jax.ShapeDtypeStruct((_B, _B), jnp.float32),
        mesh=mesh,
        compiler_params=pltpu.CompilerParams(needs_layout_passes=False),
        scratch_types=[
            pltpu.VMEM((_D, _B), jnp.float32),
            pltpu.VMEM((_RPW * _D,), jnp.float32),
            pltpu.VMEM((_B,), jnp.int32),
            pltpu.VMEM((_RPW, _B), jnp.float32),
        ],
    )(et, e_flat, labels)
    return out.reshape(-1)


# speedup vs baseline: 4.3136x; 2.1612x over previous
import jax
import jax.numpy as jnp
from jax import lax
from jax.experimental import pallas as pl
from jax.experimental.pallas import tpu as pltpu, tpu_sc as plsc

_B = 1024
_D = 32
_L = 16
_NC = 2
_NS = 16
_NW = _NC * _NS
_RPW = _B // _NW
_NCHUNK = _B // _L
_TCBLK = 128


def _d2_kernel(a_ref, e_ref, out_ref):
    a = a_ref[...]
    e = e_ref[...]
    g = lax.dot_general(a, e, dimension_numbers=(((1,), (1,)), ((), ())),
                        preferred_element_type=jnp.float32)
    na = jnp.sum(a * a, axis=1, keepdims=True)
    ne = jnp.sum(e * e, axis=1).reshape(1, _B)
    out_ref[...] = jnp.maximum(na + ne - 2.0 * g, 0.0)


def _newton_sqrt(x):
    i = lax.bitcast_convert_type(x, jnp.int32)
    i = jnp.int32(0x5F3759DF) - lax.shift_right_arithmetic(i, 1)
    y = lax.bitcast_convert_type(i, jnp.float32)
    for _ in range(2):
        y = y * (1.5 - 0.5 * x * y * y)
    return x * y


def _splat(ref, idx):
    return plsc.load_gather(ref, [jnp.full((_L,), idx, jnp.int32)])


def _sc_body(d2_hbm, lab_hbm, out_hbm, d2_v, lab_v, out_v):
    wid = lax.axis_index("s") * _NC + lax.axis_index("c")
    base = wid * _RPW
    pltpu.sync_copy(d2_hbm.at[pl.ds(base, _RPW)], d2_v)
    pltpu.sync_copy(lab_hbm, lab_v)

    @plsc.parallel_loop(0, _RPW)
    def row_body(r):
        lab_r = _splat(lab_v, base + r)

        @plsc.parallel_loop(0, _NCHUNK, unroll=8)
        def chunk_body(cc):
            c0 = cc * _L
            dist = _newton_sqrt(d2_v[r, pl.ds(c0, _L)] + 1e-12)
            eq = jnp.where(lab_v[pl.ds(c0, _L)] == lab_r, 1.0, 0.0)
            diff = eq - dist
            out_v[r, pl.ds(c0, _L)] = diff * diff

    pltpu.sync_copy(out_v, out_hbm.at[pl.ds(base, _RPW)])


def kernel(embeddings, labels):
    labels = labels.astype(jnp.int32)
    d2 = pl.pallas_call(
        _d2_kernel,
        grid=(_B // _TCBLK,),
        in_specs=[
            pl.BlockSpec((_TCBLK, _D), lambda i: (i, 0)),
            pl.BlockSpec((_B, _D), lambda i: (0, 0)),
        ],
        out_specs=pl.BlockSpec((_TCBLK, _B), lambda i: (i, 0)),
        out_shape=jax.ShapeDtypeStruct((_B, _B), jnp.float32),
    )(embeddings, embeddings)
    mesh = plsc.VectorSubcoreMesh(
        core_axis_name="c", subcore_axis_name="s",
        num_cores=_NC, num_subcores=_NS)
    out = pl.kernel(
        _sc_body,
        out_type=jax.ShapeDtypeStruct((_B, _B), jnp.float32),
        mesh=mesh,
        compiler_params=pltpu.CompilerParams(needs_layout_passes=False),
        scratch_types=[
            pltpu.VMEM((_RPW, _B), jnp.float32),
            pltpu.VMEM((_B,), jnp.int32),
            pltpu.VMEM((_RPW, _B), jnp.float32),
        ],
    )(d2, labels)
    return out.reshape(-1)
